# baseline (speedup 1.0000x reference)
import functools

import jax
import jax.numpy as jnp
from jax import lax
from jax.experimental import pallas as pl
from jax.experimental.pallas import tpu as pltpu
from jax.experimental.pallas import tpu_sc as plsc

N = 10000
E = 320000
D = 128

NC = 2
NS = 16
NW = NC * NS
C = 128
NCH = 79
EPW = NCH * C
EPAD = NW * EPW
NP = 10240
RPT = NP // NS
LANES = 16

_mesh = plsc.VectorSubcoreMesh(core_axis_name="c", subcore_axis_name="s")


@functools.partial(
    pl.kernel,
    out_type=jax.ShapeDtypeStruct((NC, NP, D), jnp.float32),
    mesh=_mesh,
    compiler_params=pltpu.CompilerParams(needs_layout_passes=False,
                                        use_tc_tiling_on_sc=False),
    scratch_types=[
        pltpu.VMEM((3, 1, C), jnp.int32),
        pltpu.VMEM((3, 1, C), jnp.int32),
        pltpu.VMEM((3, 1, C), jnp.float32),
        pltpu.VMEM((2, C, D), jnp.float32),
        pltpu.VMEM_SHARED((NP, D), jnp.float32),
        pltpu.SemaphoreType.DMA,
        pltpu.SemaphoreType.DMA,
        pltpu.SemaphoreType.DMA,
        pltpu.SemaphoreType.DMA,
        pltpu.SemaphoreType.DMA,
        pltpu.SemaphoreType.DMA,
        pltpu.SemaphoreType.DMA,
    ],
)
def _spmm(col_hbm, row_hbm, w_hbm, x_hbm, out_hbm,
          col_v, row_v, w_v, rows_f, acc,
          gsem0, gsem1, ssem0, ssem1, msem0, msem1, msem2):
    cid = lax.axis_index("c")
    sid = lax.axis_index("s")
    gid = cid * NS + sid
    gsem = (gsem0, gsem1)
    ssem = (ssem0, ssem1)
    msem = (msem0, msem1, msem2)

    zero = jnp.zeros((LANES,), jnp.float32)

    @pl.loop(0, C)
    def _zero_fill(r):
        for k in range(D // LANES):
            rows_f[0, r, pl.ds(k * LANES, LANES)] = zero

    for t in range(RPT // C):
        pltpu.sync_copy(rows_f.at[0], acc.at[pl.ds(sid * RPT + t * C, C)])
    plsc.subcore_barrier()

    def issue_meta(i, m):
        pltpu.async_copy(col_hbm.at[gid, i], col_v.at[m], msem[m])
        pltpu.async_copy(row_hbm.at[gid, i], row_v.at[m], msem[m])
        pltpu.async_copy(w_hbm.at[gid, i], w_v.at[m], msem[m])

    def wait_meta(i, m):
        pltpu.make_async_copy(col_hbm.at[gid, i], col_v.at[m], msem[m]).wait()
        pltpu.make_async_copy(row_hbm.at[gid, i], row_v.at[m], msem[m]).wait()
        pltpu.make_async_copy(w_hbm.at[gid, i], w_v.at[m], msem[m]).wait()

    def issue_gather(m, r):
        pltpu.async_copy(x_hbm.at[col_v.at[m, 0]], rows_f.at[r], gsem[r])

    def wait_gather(m, r):
        pltpu.make_async_copy(x_hbm.at[col_v.at[m, 0]], rows_f.at[r],
                              gsem[r]).wait()

    def issue_scatter(m, r):
        pltpu.async_copy(rows_f.at[r], acc.at[row_v.at[m, 0]], ssem[r],
                         add=True)

    def wait_scatter(m, r):
        pltpu.make_async_copy(rows_f.at[r], acc.at[row_v.at[m, 0]],
                              ssem[r]).wait()

    def scale(m, r):
        zz = jnp.zeros((LANES,), jnp.int32)
        mm = jnp.full((LANES,), m, jnp.int32)

        @pl.loop(0, C, unroll=2)
        def _scale(e):
            we = jnp.full((LANES,), e, jnp.int32)
            wspl = plsc.load_gather(w_v, [mm, zz, we])
            for k in range(D // LANES):
                sl = pl.ds(k * LANES, LANES)
                rows_f[r, e, sl] = rows_f[r, e, sl] * wspl

    def body(i, r, m, last_meta=False, last_gather=False):
        rr = 1 - r
        m1 = (m + 1) % 3
        m2 = (m + 2) % 3
        wait_scatter(m2, rr)
        if not last_gather:
            wait_meta(i + 1, m1)
            issue_gather(m1, rr)
        if not last_meta:
            issue_meta(i + 2, m2)
        wait_gather(m, r)
        scale(m, r)
        issue_scatter(m, r)

    issue_meta(0, 0)
    issue_meta(1, 1)
    wait_meta(0, 0)
    issue_gather(0, 0)
    wait_meta(1, 1)
    issue_gather(1, 1)
    issue_meta(2, 2)
    wait_gather(0, 0)
    scale(0, 0)
    issue_scatter(0, 0)

    @pl.loop(0, (NCH - 7) // 6)
    def _steady(t):
        i = 6 * t + 1
        body(i, 1, 1)
        body(i + 1, 0, 2)
        body(i + 2, 1, 0)
        body(i + 3, 0, 1)
        body(i + 4, 1, 2)
        body(i + 5, 0, 0)

    body(NCH - 6, 1, 1)
    body(NCH - 5, 0, 2)
    body(NCH - 4, 1, 0)
    body(NCH - 3, 0, 1)
    body(NCH - 2, 1, 2, last_meta=True)
    body(NCH - 1, 0, 0, last_meta=True, last_gather=True)
    wait_scatter(0, 0)

    plsc.subcore_barrier()
    pltpu.sync_copy(acc.at[pl.ds(sid * RPT, RPT)],
                    out_hbm.at[cid, pl.ds(sid * RPT, RPT)])


BR = 1000


def _combine_body(alpha_ref, beta_ref, part_ref, x0_ref, w_ref, out_ref):
    a = alpha_ref[0]
    b = beta_ref[0]
    agg = part_ref[0] + part_ref[1]
    h = a * agg + (1.0 - a) * x0_ref[...]
    hw = lax.dot_general(h, w_ref[...], (((1,), (1,)), ((), ())),
                         preferred_element_type=jnp.float32)
    out_ref[...] = b * hw + (1.0 - b) * h


_combine = pl.pallas_call(
    _combine_body,
    grid=(N // BR,),
    in_specs=[
        pl.BlockSpec(memory_space=pltpu.SMEM),
        pl.BlockSpec(memory_space=pltpu.SMEM),
        pl.BlockSpec((NC, BR, D), lambda i: (0, i, 0)),
        pl.BlockSpec((BR, D), lambda i: (i, 0)),
        pl.BlockSpec((D, D), lambda i: (0, 0)),
    ],
    out_specs=pl.BlockSpec((BR, D), lambda i: (i, 0)),
    out_shape=jax.ShapeDtypeStruct((N, D), jnp.float32),
)


def kernel(x, edge_index, edge_weight, x_0, alpha, beta, W):
    pad = EPAD - E
    row = jnp.pad(edge_index[0], (0, pad)).reshape(NW, NCH, 1, C)
    col = jnp.pad(edge_index[1], (0, pad)).reshape(NW, NCH, 1, C)
    w3 = jnp.pad(edge_weight, (0, pad)).reshape(NW, NCH, 1, C)
    part = _spmm(col, row, w3, x)
    a = jnp.reshape(alpha, (1,)).astype(jnp.float32)
    b = jnp.reshape(beta, (1,)).astype(jnp.float32)
    return _combine(a, b, part, x_0, W)


# speedup vs baseline: 1.7194x; 1.7194x over previous
import functools

import jax
import jax.numpy as jnp
from jax import lax
from jax.experimental import pallas as pl
from jax.experimental.pallas import tpu as pltpu
from jax.experimental.pallas import tpu_sc as plsc

N = 10000
E = 320000
D = 128

NC = 2
NS = 16
NW = NC * NS
EPW = E // NW
C = 80
NCH = EPW // C
NP = 10240
RPT = NP // NS
LANES = 16

_mesh = plsc.VectorSubcoreMesh(core_axis_name="c", subcore_axis_name="s")


@functools.partial(
    pl.kernel,
    out_type=jax.ShapeDtypeStruct((NC, NP, D), jnp.float32),
    mesh=_mesh,
    compiler_params=pltpu.CompilerParams(needs_layout_passes=False,
                                        use_tc_tiling_on_sc=False),
    scratch_types=[
        pltpu.VMEM((NCH, C), jnp.int32),
        pltpu.VMEM((2, 1, C), jnp.int32),
        pltpu.VMEM((2, 1, C), jnp.float32),
        pltpu.VMEM((2, C, D), jnp.float32),
        pltpu.VMEM((C, D), jnp.float32),
        pltpu.VMEM_SHARED((NP, D), jnp.float32),
        pltpu.SemaphoreType.DMA,
        pltpu.SemaphoreType.DMA,
        pltpu.SemaphoreType.DMA,
        pltpu.SemaphoreType.DMA,
        pltpu.SemaphoreType.DMA,
        pltpu.SemaphoreType.DMA,
    ],
)
def _spmm(col_hbm, row_hbm, w_hbm, x_hbm, out_hbm,
          col_v, row_v, w_v, rows_v, zbuf, acc,
          gsem0, gsem1, ssem0, ssem1, msem0, msem1):
    cid = lax.axis_index("c")
    sid = lax.axis_index("s")
    gid = cid * NS + sid
    gsem = (gsem0, gsem1)
    ssem = (ssem0, ssem1)
    msem = (msem0, msem1)

    def issue_meta(i, b):
        pltpu.async_copy(row_hbm.at[gid, i], row_v.at[b], msem[b])
        pltpu.async_copy(w_hbm.at[gid, i], w_v.at[b], msem[b])

    def wait_meta(i, b):
        pltpu.make_async_copy(row_hbm.at[gid, i], row_v.at[b], msem[b]).wait()
        pltpu.make_async_copy(w_hbm.at[gid, i], w_v.at[b], msem[b]).wait()

    def issue_gather(i, b):
        pltpu.async_copy(x_hbm.at[col_v.at[i]], rows_v.at[b], gsem[b])

    def wait_gather(i, b):
        pltpu.make_async_copy(x_hbm.at[col_v.at[i]], rows_v.at[b],
                              gsem[b]).wait()

    def issue_scatter(b):
        pltpu.async_copy(rows_v.at[b], acc.at[row_v.at[b, 0]], ssem[b],
                         add=True)

    def wait_scatter(b):
        pltpu.make_async_copy(rows_v.at[b], acc.at[row_v.at[b, 0]],
                              ssem[b]).wait()

    def scale(b):
        @pl.loop(0, C, unroll=4)
        def _scale(e):
            bb0 = jnp.full((LANES,), b, jnp.int32)
            zz0 = jnp.zeros((LANES,), jnp.int32)
            we = jnp.full((LANES,), e, jnp.int32)
            wspl = plsc.load_gather(w_v, [bb0, zz0, we])
            for k in range(D // LANES):
                sl = pl.ds(k * LANES, LANES)
                rows_v[b, e, sl] = rows_v[b, e, sl] * wspl

    pltpu.sync_copy(col_hbm.at[gid], col_v)
    issue_meta(0, 0)
    issue_gather(0, 0)
    issue_meta(1, 1)
    issue_gather(1, 1)

    zero = jnp.zeros((LANES,), jnp.float32)

    @pl.loop(0, C)
    def _zero_fill(r):
        for k in range(D // LANES):
            zbuf[r, pl.ds(k * LANES, LANES)] = zero

    for t in range(RPT // C):
        pltpu.sync_copy(zbuf, acc.at[pl.ds(sid * RPT + t * C, C)])
    plsc.subcore_barrier()

    wait_gather(0, 0)
    wait_meta(0, 0)
    scale(0)
    issue_scatter(0)

    def body(i, b):
        bb = 1 - b
        wait_scatter(bb)
        issue_meta(i + 1, bb)
        issue_gather(i + 1, bb)
        wait_gather(i, b)
        wait_meta(i, b)
        scale(b)
        issue_scatter(b)

    @pl.loop(0, (NCH - 3) // 2)
    def _steady(t):
        i = 2 * t + 1
        body(i, 1)
        body(i + 1, 0)

    body(NCH - 2, 1)
    wait_scatter(1)
    wait_gather(NCH - 1, 0)
    wait_meta(NCH - 1, 0)
    scale(0)
    issue_scatter(0)
    wait_scatter(0)

    plsc.subcore_barrier()
    pltpu.sync_copy(acc.at[pl.ds(sid * RPT, RPT)],
                    out_hbm.at[cid, pl.ds(sid * RPT, RPT)])


BR = 1000


def _combine_body(alpha_ref, beta_ref, part_ref, x0_ref, w_ref, out_ref):
    a = alpha_ref[0]
    b = beta_ref[0]
    agg = part_ref[0] + part_ref[1]
    h = a * agg + (1.0 - a) * x0_ref[...]
    hw = lax.dot_general(h, w_ref[...], (((1,), (1,)), ((), ())),
                         preferred_element_type=jnp.float32)
    out_ref[...] = b * hw + (1.0 - b) * h


_combine = pl.pallas_call(
    _combine_body,
    grid=(N // BR,),
    in_specs=[
        pl.BlockSpec(memory_space=pltpu.SMEM),
        pl.BlockSpec(memory_space=pltpu.SMEM),
        pl.BlockSpec((NC, BR, D), lambda i: (0, i, 0)),
        pl.BlockSpec((BR, D), lambda i: (i, 0)),
        pl.BlockSpec((D, D), lambda i: (0, 0)),
    ],
    out_specs=pl.BlockSpec((BR, D), lambda i: (i, 0)),
    out_shape=jax.ShapeDtypeStruct((N, D), jnp.float32),
)


def kernel(x, edge_index, edge_weight, x_0, alpha, beta, W):
    row = edge_index[0].reshape(NW, NCH, 1, C)
    col = edge_index[1].reshape(NW, NCH, C)
    w3 = edge_weight.reshape(NW, NCH, 1, C)
    part = _spmm(col, row, w3, x)
    a = jnp.reshape(alpha, (1,)).astype(jnp.float32)
    b = jnp.reshape(beta, (1,)).astype(jnp.float32)
    return _combine(a, b, part, x_0, W)


# speedup vs baseline: 1.9687x; 1.1450x over previous
import functools

import jax
import jax.numpy as jnp
from jax import lax
from jax.experimental import pallas as pl
from jax.experimental.pallas import tpu as pltpu
from jax.experimental.pallas import tpu_sc as plsc

N = 10000
E = 320000
D = 128

NC = 2
NS = 16
NW = NC * NS
EPW = E // NW
C = 80
NCH = EPW // C
NP = 10240
RPT = NP // NS
LANES = 16

_mesh = plsc.VectorSubcoreMesh(core_axis_name="c", subcore_axis_name="s")


@functools.partial(
    pl.kernel,
    out_type=jax.ShapeDtypeStruct((NC, NP, D), jnp.float32),
    mesh=_mesh,
    compiler_params=pltpu.CompilerParams(needs_layout_passes=False,
                                        use_tc_tiling_on_sc=False),
    scratch_types=[
        pltpu.VMEM((NCH, C), jnp.int32),
        pltpu.VMEM((3, 1, C), jnp.int32),
        pltpu.VMEM((3, 1, C), jnp.float32),
        pltpu.VMEM((3, C, D), jnp.float32),
        pltpu.VMEM_SHARED((NP, D), jnp.float32),
        pltpu.SemaphoreType.DMA,
        pltpu.SemaphoreType.DMA,
        pltpu.SemaphoreType.DMA,
        pltpu.SemaphoreType.DMA,
        pltpu.SemaphoreType.DMA,
        pltpu.SemaphoreType.DMA,
        pltpu.SemaphoreType.DMA,
        pltpu.SemaphoreType.DMA,
        pltpu.SemaphoreType.DMA,
    ],
)
def _spmm(col_hbm, row_hbm, w_hbm, x_hbm, out_hbm,
          col_v, row_v, w_v, rows_v, acc,
          gsem0, gsem1, gsem2, ssem0, ssem1, ssem2, msem0, msem1, msem2):
    cid = lax.axis_index("c")
    sid = lax.axis_index("s")
    gid = cid * NS + sid
    gsem = (gsem0, gsem1, gsem2)
    ssem = (ssem0, ssem1, ssem2)
    msem = (msem0, msem1, msem2)

    def issue_meta(i, b):
        pltpu.async_copy(row_hbm.at[gid, i], row_v.at[b], msem[b])
        pltpu.async_copy(w_hbm.at[gid, i], w_v.at[b], msem[b])

    def wait_meta(i, b):
        pltpu.make_async_copy(row_hbm.at[gid, i], row_v.at[b], msem[b]).wait()
        pltpu.make_async_copy(w_hbm.at[gid, i], w_v.at[b], msem[b]).wait()

    def issue_gather(i, b):
        pltpu.async_copy(x_hbm.at[col_v.at[i]], rows_v.at[b], gsem[b])

    def wait_gather(i, b):
        pltpu.make_async_copy(x_hbm.at[col_v.at[i]], rows_v.at[b],
                              gsem[b]).wait()

    def issue_scatter(b):
        pltpu.async_copy(rows_v.at[b], acc.at[row_v.at[b, 0]], ssem[b],
                         add=True)

    def wait_scatter(b):
        pltpu.make_async_copy(rows_v.at[b], acc.at[row_v.at[b, 0]],
                              ssem[b]).wait()

    def scale(b):
        @pl.loop(0, C, unroll=4)
        def _scale(e):
            bb0 = jnp.full((LANES,), b, jnp.int32)
            zz0 = jnp.zeros((LANES,), jnp.int32)
            we = jnp.full((LANES,), e, jnp.int32)
            wspl = plsc.load_gather(w_v, [bb0, zz0, we])
            for k in range(D // LANES):
                sl = pl.ds(k * LANES, LANES)
                rows_v[b, e, sl] = rows_v[b, e, sl] * wspl

    pltpu.sync_copy(col_hbm.at[gid], col_v)
    issue_meta(0, 0)
    issue_gather(0, 0)
    issue_meta(1, 1)
    issue_gather(1, 1)

    zero = jnp.zeros((LANES,), jnp.float32)

    @pl.loop(0, C)
    def _zero_fill(r):
        for k in range(D // LANES):
            rows_v[2, r, pl.ds(k * LANES, LANES)] = zero

    for t in range(RPT // C):
        pltpu.sync_copy(rows_v.at[2], acc.at[pl.ds(sid * RPT + t * C, C)])
    plsc.subcore_barrier()

    def body(i, b, first=False, last=False):
        b1 = (b + 1) % 3
        if not first:
            wait_scatter(b1)
        if not last:
            issue_meta(i + 1, b1)
            issue_gather(i + 1, b1)
        wait_gather(i, b)
        wait_meta(i, b)
        scale(b)
        issue_scatter(b)

    wait_gather(0, 0)
    wait_meta(0, 0)
    scale(0)
    issue_scatter(0)
    body(1, 1, first=True)

    @pl.loop(0, (NCH - 5) // 3)
    def _steady(t):
        i = 3 * t + 2
        body(i, 2)
        body(i + 1, 0)
        body(i + 2, 1)

    body(NCH - 3, 2)
    body(NCH - 2, 0)
    body(NCH - 1, 1, last=True)
    wait_scatter(0)
    wait_scatter(1)

    plsc.subcore_barrier()
    pltpu.sync_copy(acc.at[pl.ds(sid * RPT, RPT)],
                    out_hbm.at[cid, pl.ds(sid * RPT, RPT)])


BR = 1000


def _combine_body(alpha_ref, beta_ref, part_ref, x0_ref, w_ref, out_ref):
    a = alpha_ref[0]
    b = beta_ref[0]
    agg = part_ref[0] + part_ref[1]
    h = a * agg + (1.0 - a) * x0_ref[...]
    hw = lax.dot_general(h, w_ref[...], (((1,), (1,)), ((), ())),
                         preferred_element_type=jnp.float32)
    out_ref[...] = b * hw + (1.0 - b) * h


_combine = pl.pallas_call(
    _combine_body,
    grid=(N // BR,),
    in_specs=[
        pl.BlockSpec(memory_space=pltpu.SMEM),
        pl.BlockSpec(memory_space=pltpu.SMEM),
        pl.BlockSpec((NC, BR, D), lambda i: (0, i, 0)),
        pl.BlockSpec((BR, D), lambda i: (i, 0)),
        pl.BlockSpec((D, D), lambda i: (0, 0)),
    ],
    out_specs=pl.BlockSpec((BR, D), lambda i: (i, 0)),
    out_shape=jax.ShapeDtypeStruct((N, D), jnp.float32),
)


def kernel(x, edge_index, edge_weight, x_0, alpha, beta, W):
    row = edge_index[0].reshape(NW, NCH, 1, C)
    col = edge_index[1].reshape(NW, NCH, C)
    w3 = edge_weight.reshape(NW, NCH, 1, C)
    part = _spmm(col, row, w3, x)
    a = jnp.reshape(alpha, (1,)).astype(jnp.float32)
    b = jnp.reshape(beta, (1,)).astype(jnp.float32)
    return _combine(a, b, part, x_0, W)


# speedup vs baseline: 2.0027x; 1.0173x over previous
import functools

import jax
import jax.numpy as jnp
from jax import lax
from jax.experimental import pallas as pl
from jax.experimental.pallas import tpu as pltpu
from jax.experimental.pallas import tpu_sc as plsc

N = 10000
E = 320000
D = 128

NC = 2
NS = 16
NW = NC * NS
EPW = E // NW
C = 80
NCH = EPW // C
NP = 10240
RPT = NP // NS
LANES = 16

_mesh = plsc.VectorSubcoreMesh(core_axis_name="c", subcore_axis_name="s")


@functools.partial(
    pl.kernel,
    out_type=jax.ShapeDtypeStruct((NC, NP, D), jnp.float32),
    mesh=_mesh,
    compiler_params=pltpu.CompilerParams(needs_layout_passes=False,
                                        use_tc_tiling_on_sc=False),
    scratch_types=[
        pltpu.VMEM((NCH, C), jnp.int32),
        pltpu.VMEM((3, 1, C), jnp.int32),
        pltpu.VMEM((3, 1, C), jnp.float32),
        pltpu.VMEM((3, C, D), jnp.float32),
        pltpu.VMEM_SHARED((NP, D), jnp.float32),
        pltpu.SemaphoreType.DMA,
        pltpu.SemaphoreType.DMA,
        pltpu.SemaphoreType.DMA,
        pltpu.SemaphoreType.DMA,
        pltpu.SemaphoreType.DMA,
        pltpu.SemaphoreType.DMA,
        pltpu.SemaphoreType.DMA,
        pltpu.SemaphoreType.DMA,
        pltpu.SemaphoreType.DMA,
    ],
)
def _spmm(col_hbm, row_hbm, w_hbm, x_hbm, out_hbm,
          col_v, row_v, w_v, rows_v, acc,
          gsem0, gsem1, gsem2, ssem0, ssem1, ssem2, msem0, msem1, msem2):
    cid = lax.axis_index("c")
    sid = lax.axis_index("s")
    gid = cid * NS + sid
    gsem = (gsem0, gsem1, gsem2)
    ssem = (ssem0, ssem1, ssem2)
    msem = (msem0, msem1, msem2)

    def issue_meta(i, b):
        pltpu.async_copy(row_hbm.at[gid, i], row_v.at[b], msem[b])
        pltpu.async_copy(w_hbm.at[gid, i], w_v.at[b], msem[b])

    def wait_meta(i, b):
        pltpu.make_async_copy(row_hbm.at[gid, i], row_v.at[b], msem[b]).wait()
        pltpu.make_async_copy(w_hbm.at[gid, i], w_v.at[b], msem[b]).wait()

    def issue_gather(i, b):
        pltpu.async_copy(x_hbm.at[col_v.at[i]], rows_v.at[b], gsem[b])

    def wait_gather(i, b):
        pltpu.make_async_copy(x_hbm.at[col_v.at[i]], rows_v.at[b],
                              gsem[b]).wait()

    def issue_scatter(b):
        pltpu.async_copy(rows_v.at[b], acc.at[row_v.at[b, 0]], ssem[b],
                         add=True)

    def wait_scatter(b):
        pltpu.make_async_copy(rows_v.at[b], acc.at[row_v.at[b, 0]],
                              ssem[b]).wait()

    def scale(b):
        @pl.loop(0, C, unroll=4)
        def _scale(e):
            bb0 = jnp.full((LANES,), b, jnp.int32)
            zz0 = jnp.zeros((LANES,), jnp.int32)
            we = jnp.full((LANES,), e, jnp.int32)
            wspl = plsc.load_gather(w_v, [bb0, zz0, we])
            for k in range(D // LANES):
                sl = pl.ds(k * LANES, LANES)
                rows_v[b, e, sl] = rows_v[b, e, sl] * wspl

    pltpu.sync_copy(col_hbm.at[gid], col_v)
    issue_meta(0, 0)
    issue_gather(0, 0)
    issue_meta(1, 1)
    issue_gather(1, 1)

    zero = jnp.zeros((LANES,), jnp.float32)

    @pl.loop(0, C)
    def _zero_fill(r):
        for k in range(D // LANES):
            rows_v[2, r, pl.ds(k * LANES, LANES)] = zero

    for t in range(RPT // C):
        pltpu.sync_copy(rows_v.at[2], acc.at[pl.ds(sid * RPT + t * C, C)])
    plsc.subcore_barrier()

    def body(i, b, first=False, last=False):
        b1 = (b + 1) % 3
        if not first:
            wait_scatter(b1)
        if not last:
            issue_meta(i + 1, b1)
            issue_gather(i + 1, b1)
        wait_gather(i, b)
        wait_meta(i, b)
        scale(b)
        issue_scatter(b)

    wait_gather(0, 0)
    wait_meta(0, 0)
    scale(0)
    issue_scatter(0)
    body(1, 1, first=True)

    @pl.loop(0, (NCH - 5) // 3)
    def _steady(t):
        i = 3 * t + 2
        body(i, 2)
        body(i + 1, 0)
        body(i + 2, 1)

    body(NCH - 3, 2)
    body(NCH - 2, 0)
    body(NCH - 1, 1, last=True)
    wait_scatter(0)
    wait_scatter(1)

    plsc.subcore_barrier()
    pltpu.sync_copy(acc.at[pl.ds(sid * RPT, RPT)],
                    out_hbm.at[cid, pl.ds(sid * RPT, RPT)])


BR = 2000


def _combine_body(alpha_ref, beta_ref, part_ref, x0_ref, w_ref, out_ref):
    a = alpha_ref[0]
    b = beta_ref[0]
    agg = part_ref[0] + part_ref[1]
    h = a * agg + (1.0 - a) * x0_ref[...]
    hw = lax.dot_general(h, w_ref[...], (((1,), (1,)), ((), ())),
                         preferred_element_type=jnp.float32)
    out_ref[...] = b * hw + (1.0 - b) * h


_combine = pl.pallas_call(
    _combine_body,
    grid=(N // BR,),
    in_specs=[
        pl.BlockSpec(memory_space=pltpu.SMEM),
        pl.BlockSpec(memory_space=pltpu.SMEM),
        pl.BlockSpec((NC, BR, D), lambda i: (0, i, 0)),
        pl.BlockSpec((BR, D), lambda i: (i, 0)),
        pl.BlockSpec((D, D), lambda i: (0, 0)),
    ],
    out_specs=pl.BlockSpec((BR, D), lambda i: (i, 0)),
    out_shape=jax.ShapeDtypeStruct((N, D), jnp.float32),
)


def kernel(x, edge_index, edge_weight, x_0, alpha, beta, W):
    row = edge_index[0].reshape(NW, NCH, 1, C)
    col = edge_index[1].reshape(NW, NCH, C)
    w3 = edge_weight.reshape(NW, NCH, 1, C)
    part = _spmm(col, row, w3, x)
    a = jnp.reshape(alpha, (1,)).astype(jnp.float32)
    b = jnp.reshape(beta, (1,)).astype(jnp.float32)
    return _combine(a, b, part, x_0, W)
